# baseline (speedup 1.0000x reference)
import functools

import jax
import jax.numpy as jnp
from jax.experimental import pallas as pl
from jax.experimental.pallas import tpu as pltpu


def _mean_tree_kernel(x_ref, o_ref, *, num_neigh, inv_n):
    vals = [x_ref[:, n, :].astype(jnp.float32) for n in range(num_neigh)]
    while len(vals) > 1:
        nxt = [vals[i] + vals[i + 1] for i in range(0, len(vals) - 1, 2)]
        if len(vals) % 2:
            nxt.append(vals[-1])
        vals = nxt
    o_ref[...] = (vals[0] * inv_n).astype(o_ref.dtype)


def _pick_tile(num_src, row_bytes):
    target = 8 << 20
    tile = max(8, min(num_src, target // max(row_bytes, 1)))
    tile -= tile % 8
    t = tile
    while t >= 8:
        if num_src % t == 0:
            return t
        t -= 8
    return max(tile, 8)


def kernel(neighbor_feature):
    num_src, num_neigh, input_dim = neighbor_feature.shape
    dtype = neighbor_feature.dtype
    itemsize = jnp.dtype(dtype).itemsize

    row_bytes = num_neigh * input_dim * itemsize
    tile = _pick_tile(num_src, row_bytes)
    grid = (pl.cdiv(num_src, tile),)

    kfn = functools.partial(
        _mean_tree_kernel, num_neigh=num_neigh, inv_n=1.0 / float(num_neigh))

    in_bytes = tile * row_bytes
    out_bytes = tile * input_dim * itemsize
    vmem_limit = int(min(100 << 20, 2 * in_bytes + 2 * out_bytes + (4 << 20)))

    return pl.pallas_call(
        kfn,
        out_shape=jax.ShapeDtypeStruct((num_src, input_dim), dtype),
        grid=grid,
        in_specs=[pl.BlockSpec((tile, num_neigh, input_dim),
                               lambda i: (i, 0, 0))],
        out_specs=pl.BlockSpec((tile, input_dim), lambda i: (i, 0)),
        compiler_params=pltpu.CompilerParams(
            dimension_semantics=("parallel",),
            vmem_limit_bytes=vmem_limit,
        ),
        cost_estimate=pl.CostEstimate(
            flops=num_src * num_neigh * input_dim,
            transcendentals=0,
            bytes_accessed=num_src * (num_neigh + 1) * input_dim * itemsize,
        ),
    )(neighbor_feature)


# speedup vs baseline: 1.1686x; 1.1686x over previous
import functools

import jax
import jax.numpy as jnp
from jax.experimental import pallas as pl
from jax.experimental.pallas import tpu as pltpu


def _mean_tree_kernel(x_ref, o_ref, *, num_neigh, inv_n):
    del num_neigh
    s = jnp.sum(x_ref[...].astype(jnp.float32), axis=1)
    o_ref[...] = (s * inv_n).astype(o_ref.dtype)


def _pick_tile(num_src, row_bytes):
    target = 8 << 20
    tile = max(8, min(num_src, target // max(row_bytes, 1)))
    tile -= tile % 8
    t = tile
    while t >= 8:
        if num_src % t == 0:
            return t
        t -= 8
    return max(tile, 8)


def kernel(neighbor_feature):
    num_src, num_neigh, input_dim = neighbor_feature.shape
    dtype = neighbor_feature.dtype
    itemsize = jnp.dtype(dtype).itemsize

    row_bytes = num_neigh * input_dim * itemsize
    tile = _pick_tile(num_src, row_bytes)
    grid = (pl.cdiv(num_src, tile),)

    kfn = functools.partial(
        _mean_tree_kernel, num_neigh=num_neigh, inv_n=1.0 / float(num_neigh))

    in_bytes = tile * row_bytes
    out_bytes = tile * input_dim * itemsize
    vmem_limit = int(min(100 << 20, 2 * in_bytes + 2 * out_bytes + (4 << 20)))

    return pl.pallas_call(
        kfn,
        out_shape=jax.ShapeDtypeStruct((num_src, input_dim), dtype),
        grid=grid,
        in_specs=[pl.BlockSpec((tile, num_neigh, input_dim),
                               lambda i: (i, 0, 0))],
        out_specs=pl.BlockSpec((tile, input_dim), lambda i: (i, 0)),
        compiler_params=pltpu.CompilerParams(
            dimension_semantics=("parallel",),
            vmem_limit_bytes=vmem_limit,
        ),
        cost_estimate=pl.CostEstimate(
            flops=num_src * num_neigh * input_dim,
            transcendentals=0,
            bytes_accessed=num_src * (num_neigh + 1) * input_dim * itemsize,
        ),
    )(neighbor_feature)
